# baseline (speedup 1.0000x reference)
import functools

import jax
import jax.numpy as jnp
from jax import lax
from jax.experimental import pallas as pl
from jax.experimental.pallas import tpu as pltpu
from jax.experimental.pallas import tpu_sc as plsc

B, N, D = 1024, 200, 128
K = 50
BB = 64

NW = 32
ROWS = B * K
RPW = ROWS // NW
CHUNK = 80
NCH = RPW // CHUNK


def _topk_idx_body(x_ref, idx_ref):
    pid = pl.program_id(0)
    x3 = x_ref[...]
    norms = jnp.sum(x3 * x3, axis=2)
    j_row = lax.broadcasted_iota(jnp.int32, (BB, N), 1)
    p_row = lax.broadcasted_iota(jnp.int32, (BB, K), 1)
    cur = norms
    acc = jnp.zeros((BB, K), jnp.float32)
    for p in range(K):
        m = jnp.max(cur, axis=1, keepdims=True)
        cand = jnp.where(cur == m, j_row, N)
        jstar = jnp.min(cand, axis=1, keepdims=True)
        cur = jnp.where(j_row == jstar, -1.0, cur)
        acc = acc + jnp.where(p_row == p, jstar.astype(jnp.float32), 0.0)
    brow = lax.broadcasted_iota(jnp.int32, (BB, K), 0)
    base_f = ((pid * BB + brow) * N).astype(jnp.float32)
    idx_ref[0] = (acc + base_f).astype(jnp.int32)


def _topk_indices(x):
    idx = pl.pallas_call(
        _topk_idx_body,
        grid=(B // BB,),
        in_specs=[pl.BlockSpec((BB, N, D), lambda i: (i, 0, 0))],
        out_specs=pl.BlockSpec((1, BB, K), lambda i: (i, 0, 0)),
        out_shape=jax.ShapeDtypeStruct((B // BB, BB, K), jnp.int32),
    )(x)
    return idx.reshape(B, K)


def _sc_gather(x2d, idx3):
    mesh = plsc.VectorSubcoreMesh(core_axis_name="c", subcore_axis_name="s")

    @functools.partial(
        pl.kernel,
        mesh=mesh,
        out_type=jax.ShapeDtypeStruct((ROWS, D), jnp.float32),
        scratch_types=[
            pltpu.VMEM((NCH, CHUNK), jnp.int32),
            pltpu.VMEM((CHUNK, D), jnp.float32),
            pltpu.VMEM((CHUNK, D), jnp.float32),
            pltpu.SemaphoreType.DMA,
            pltpu.SemaphoreType.DMA,
        ],
    )
    def gather_kernel(x_hbm, idx_hbm, out_hbm, idx_v, buf0, buf1, sem0, sem1):
        cid = lax.axis_index("c")
        sid = lax.axis_index("s")
        wid = sid * 2 + cid
        base = wid * RPW
        pltpu.sync_copy(idx_hbm.at[wid], idx_v)
        bufs = (buf0, buf1)
        sems = (sem0, sem1)
        cps = [None, None]
        cps[0] = pltpu.async_copy(x_hbm.at[idx_v.at[0]], buf0, sem0)
        for c in range(NCH):
            if c + 1 < NCH:
                nxt = (c + 1) % 2
                cps[nxt] = pltpu.async_copy(
                    x_hbm.at[idx_v.at[c + 1]], bufs[nxt], sems[nxt]
                )
            cur = c % 2
            cps[cur].wait()
            pltpu.sync_copy(
                bufs[cur], out_hbm.at[pl.ds(base + c * CHUNK, CHUNK)]
            )

    return gather_kernel(x2d, idx3)


def kernel(x):
    idx = _topk_indices(x)
    idx3 = idx.reshape(NW, NCH, CHUNK)
    out = _sc_gather(x.reshape(B * N, D), idx3)
    return out.reshape(B, K, D)


# speedup vs baseline: 1.3702x; 1.3702x over previous
import functools

import jax
import jax.numpy as jnp
from jax import lax
from jax.experimental import pallas as pl
from jax.experimental.pallas import tpu as pltpu
from jax.experimental.pallas import tpu_sc as plsc

B, N, D = 1024, 200, 128
K = 50
BB = 128

NW = 32
ROWS = B * K
RPW = ROWS // NW
CHUNK = 80
NCH = RPW // CHUNK


def _topk_idx_body(x_ref, idx_ref):
    pid = pl.program_id(0)
    x3 = x_ref[...]
    norms = jnp.sum(x3 * x3, axis=2)
    j_row = lax.broadcasted_iota(jnp.int32, (BB, N), 1)
    p_row = lax.broadcasted_iota(jnp.int32, (BB, K), 1)
    cur = norms
    acc = jnp.zeros((BB, K), jnp.float32)
    for p in range(K):
        m = jnp.max(cur, axis=1, keepdims=True)
        cand = jnp.where(cur == m, j_row, N)
        jstar = jnp.min(cand, axis=1, keepdims=True)
        cur = jnp.where(j_row == jstar, -1.0, cur)
        acc = acc + jnp.where(p_row == p, jstar.astype(jnp.float32), 0.0)
    brow = lax.broadcasted_iota(jnp.int32, (BB, K), 0)
    base_f = ((pid * BB + brow) * N).astype(jnp.float32)
    idx_ref[0] = (acc + base_f).astype(jnp.int32)


def _topk_indices(x):
    idx = pl.pallas_call(
        _topk_idx_body,
        grid=(B // BB,),
        in_specs=[pl.BlockSpec((BB, N, D), lambda i: (i, 0, 0))],
        out_specs=pl.BlockSpec((1, BB, K), lambda i: (i, 0, 0)),
        out_shape=jax.ShapeDtypeStruct((B // BB, BB, K), jnp.int32),
    )(x)
    return idx.reshape(B, K)


def _sc_gather(x2d, idx3):
    mesh = plsc.VectorSubcoreMesh(core_axis_name="c", subcore_axis_name="s")

    @functools.partial(
        pl.kernel,
        mesh=mesh,
        out_type=jax.ShapeDtypeStruct((ROWS, D), jnp.float32),
        scratch_types=[
            pltpu.VMEM((NCH, CHUNK), jnp.int32),
            pltpu.VMEM((CHUNK, D), jnp.float32),
            pltpu.VMEM((CHUNK, D), jnp.float32),
            pltpu.SemaphoreType.DMA,
            pltpu.SemaphoreType.DMA,
        ],
    )
    def gather_kernel(x_hbm, idx_hbm, out_hbm, idx_v, buf0, buf1, sem0, sem1):
        cid = lax.axis_index("c")
        sid = lax.axis_index("s")
        wid = sid * 2 + cid
        base = wid * RPW
        pltpu.sync_copy(idx_hbm.at[wid], idx_v)
        bufs = (buf0, buf1)
        sems = (sem0, sem1)
        cps = [None, None]
        cps[0] = pltpu.async_copy(x_hbm.at[idx_v.at[0]], buf0, sem0)
        for c in range(NCH):
            if c + 1 < NCH:
                nxt = (c + 1) % 2
                cps[nxt] = pltpu.async_copy(
                    x_hbm.at[idx_v.at[c + 1]], bufs[nxt], sems[nxt]
                )
            cur = c % 2
            cps[cur].wait()
            pltpu.sync_copy(
                bufs[cur], out_hbm.at[pl.ds(base + c * CHUNK, CHUNK)]
            )

    return gather_kernel(x2d, idx3)


def kernel(x):
    idx = _topk_indices(x)
    idx3 = idx.reshape(NW, NCH, CHUNK)
    out = _sc_gather(x.reshape(B * N, D), idx3)
    return out.reshape(B, K, D)


# speedup vs baseline: 1.5637x; 1.1412x over previous
import functools

import jax
import jax.numpy as jnp
from jax import lax
from jax.experimental import pallas as pl
from jax.experimental.pallas import tpu as pltpu
from jax.experimental.pallas import tpu_sc as plsc

B, N, D = 1024, 200, 128
K = 50
BB = 128

NW = 32
ROWS = B * K
RPW = ROWS // NW
CHUNK = 80
NCH = RPW // CHUNK


def _topk_idx_body(x_ref, idx_ref):
    pid = pl.program_id(0)
    x3 = x_ref[...]
    norms = jnp.sum(x3 * x3, axis=2)
    j_row = lax.broadcasted_iota(jnp.int32, (BB, N), 1)
    p_row = lax.broadcasted_iota(jnp.int32, (BB, K), 1)
    cur = norms
    acc = jnp.zeros((BB, K), jnp.float32)
    for p in range(K):
        m = jnp.max(cur, axis=1, keepdims=True)
        cand = jnp.where(cur == m, j_row, N)
        jstar = jnp.min(cand, axis=1, keepdims=True)
        cur = jnp.where(j_row == jstar, -1.0, cur)
        acc = acc + jnp.where(p_row == p, jstar.astype(jnp.float32), 0.0)
    brow = lax.broadcasted_iota(jnp.int32, (BB, K), 0)
    base_f = ((pid * BB + brow) * N).astype(jnp.float32)
    idx_ref[0] = (acc + base_f).astype(jnp.int32)


def _topk_indices(x):
    idx = pl.pallas_call(
        _topk_idx_body,
        grid=(B // BB,),
        in_specs=[pl.BlockSpec((BB, N, D), lambda i: (i, 0, 0))],
        out_specs=pl.BlockSpec((1, BB, K), lambda i: (i, 0, 0)),
        out_shape=jax.ShapeDtypeStruct((B // BB, BB, K), jnp.int32),
    )(x)
    return idx.reshape(B, K)


BPW = B // NW


def _sc_gather(x2d, idx3):
    mesh = plsc.VectorSubcoreMesh(core_axis_name="c", subcore_axis_name="s")

    @functools.partial(
        pl.kernel,
        mesh=mesh,
        out_type=jax.ShapeDtypeStruct((B, K, D), jnp.float32),
        scratch_types=[
            pltpu.VMEM((BPW, K), jnp.int32),
            pltpu.VMEM((K, D), jnp.float32),
            pltpu.VMEM((K, D), jnp.float32),
            pltpu.SemaphoreType.DMA,
            pltpu.SemaphoreType.DMA,
        ],
    )
    def gather_kernel(x_hbm, idx_hbm, out_hbm, idx_v, buf0, buf1, sem0, sem1):
        cid = lax.axis_index("c")
        sid = lax.axis_index("s")
        wid = sid * 2 + cid
        base = wid * BPW
        pltpu.sync_copy(idx_hbm.at[wid], idx_v)
        cp0 = pltpu.async_copy(x_hbm.at[idx_v.at[0]], buf0, sem0)

        def two(i, _):
            b0 = 2 * i
            nxt1 = pltpu.async_copy(x_hbm.at[idx_v.at[b0 + 1]], buf1, sem1)
            pltpu.make_async_copy(x_hbm.at[idx_v.at[b0]], buf0, sem0).wait()
            pltpu.sync_copy(buf0, out_hbm.at[base + b0])

            @pl.when(b0 + 2 < BPW)
            def _():
                pltpu.async_copy(x_hbm.at[idx_v.at[b0 + 2]], buf0, sem0)

            nxt1.wait()
            pltpu.sync_copy(buf1, out_hbm.at[base + b0 + 1])
            return 0

        lax.fori_loop(0, BPW // 2, two, 0)

    return gather_kernel(x2d, idx3)


def kernel(x):
    idx = _topk_indices(x)
    idx3 = idx.reshape(NW, BPW, K)
    out = _sc_gather(x.reshape(B * N, D), idx3)
    return out
